# baseline (speedup 1.0000x reference)
import functools
import jax
import jax.numpy as jnp
from jax import lax
from jax.experimental import pallas as pl
from jax.experimental.pallas import tpu as pltpu
from jax.experimental.pallas import tpu_sc as plsc

BATCH = 16384
EMBED_DIM = 32
NUM_WORKERS = 32
B_PER_W = BATCH // NUM_WORKERS
GATHER_CHUNK = 128
N_CHUNKS = B_PER_W // GATHER_CHUNK
LANES = 16
N_GROUPS = B_PER_W // LANES


def _gmf_body(u_idx_hbm, i_idx_hbm, u_tab_hbm, i_tab_hbm, params_hbm,
              out_hbm, u_idx_v, i_idx_v, u_rows, i_rows, params_v, out_v,
              sem):
    c = lax.axis_index("c")
    s = lax.axis_index("s")
    wid = s * 2 + c

    pltpu.sync_copy(u_idx_hbm.at[wid], u_idx_v)
    pltpu.sync_copy(i_idx_hbm.at[wid], i_idx_v)
    pltpu.sync_copy(params_hbm, params_v)

    copies = []
    for j in range(N_CHUNKS):
        copies.append(pltpu.async_copy(
            u_tab_hbm.at[u_idx_v.at[j]],
            u_rows.at[pl.ds(j * GATHER_CHUNK, GATHER_CHUNK)], sem))
        copies.append(pltpu.async_copy(
            i_tab_hbm.at[i_idx_v.at[j]],
            i_rows.at[pl.ds(j * GATHER_CHUNK, GATHER_CHUNK)], sem))
    for cp in copies:
        cp.wait()

    bias_v = params_v[pl.ds(32, LANES)]
    wv0 = params_v[pl.ds(0, LANES)]
    wv1 = params_v[pl.ds(LANES, LANES)]
    iota = lax.iota(jnp.int32, LANES)
    wbs = [(wv0 if d < LANES else wv1)
           .at[jnp.full((LANES,), d % LANES, jnp.int32)]
           .get(mode="promise_in_bounds")
           for d in range(EMBED_DIM)]

    def g_body(g, carry):
        b_vec = g * LANES + iota
        acc = jnp.zeros((LANES,), jnp.float32)
        for d in range(EMBED_DIM):
            dv = jnp.full((LANES,), d, jnp.int32)
            gu = plsc.load_gather(u_rows, [b_vec, dv])
            gi = plsc.load_gather(i_rows, [b_vec, dv])
            acc = acc + gu * gi * wbs[d]
        x = acc + bias_v
        out_v[pl.ds(g * LANES, LANES)] = 1.0 / (1.0 + jnp.exp(-x))
        return carry

    lax.fori_loop(0, N_GROUPS, g_body, 0)

    pltpu.sync_copy(out_v, out_hbm.at[pl.ds(wid * B_PER_W, B_PER_W)])


@jax.jit
def _gmf(u_idx, i_idx, user_table, item_table, params):
    mesh = plsc.VectorSubcoreMesh(core_axis_name="c", subcore_axis_name="s")
    f = pl.kernel(
        _gmf_body,
        out_type=jax.ShapeDtypeStruct((BATCH,), jnp.float32),
        mesh=mesh,
        scratch_types=[
            pltpu.VMEM((N_CHUNKS, GATHER_CHUNK), jnp.int32),
            pltpu.VMEM((N_CHUNKS, GATHER_CHUNK), jnp.int32),
            pltpu.VMEM((B_PER_W, EMBED_DIM), jnp.float32),
            pltpu.VMEM((B_PER_W, EMBED_DIM), jnp.float32),
            pltpu.VMEM((48,), jnp.float32),
            pltpu.VMEM((B_PER_W,), jnp.float32),
            pltpu.SemaphoreType.DMA,
        ],
        compiler_params=pltpu.CompilerParams(
            needs_layout_passes=False, use_tc_tiling_on_sc=False),
    )
    return f(u_idx, i_idx, user_table, item_table, params)


def kernel(user_indices, item_indices, user_table, item_table, affine_w,
           affine_b):
    u_idx = user_indices.astype(jnp.int32).reshape(
        NUM_WORKERS, N_CHUNKS, GATHER_CHUNK)
    i_idx = item_indices.astype(jnp.int32).reshape(
        NUM_WORKERS, N_CHUNKS, GATHER_CHUNK)
    params = jnp.concatenate([
        affine_w.reshape(-1).astype(jnp.float32),
        jnp.broadcast_to(affine_b.astype(jnp.float32), (LANES,)),
    ])
    return _gmf(u_idx, i_idx, user_table, item_table, params)


# speedup vs baseline: 2.4433x; 2.4433x over previous
import jax
import jax.numpy as jnp
from jax import lax
from jax.experimental import pallas as pl
from jax.experimental.pallas import tpu as pltpu
from jax.experimental.pallas import tpu_sc as plsc

BATCH = 16384
NUM_USERS = 1000000
NUM_ITEMS = 100000
LANES = 16
NTILES = 16
NGROUPS = BATCH // LANES
U_MAIN = 999936
I_MAIN = 99968
SUB = 62464
SUBC = SUB // 2
LAST_LO = 15 * SUB
LAST_C1 = U_MAIN - LAST_LO - SUBC
ICH0 = 50048
ICH1 = I_MAIN - ICH0
BIG = 50048


def _k1_body(ut_hbm, it_hbm, utail_hbm, itail_hbm, u_idx_hbm, i_idx_hbm,
             w_hbm, part_hbm,
             stage_v, q_v, acc_v, idx_v, piece_v, w_v, utail_v, itail_v,
             q_sp):
    c = lax.axis_index("c")
    t = lax.axis_index("s")
    dg = c * NTILES + t
    iota = lax.iota(jnp.int32, LANES)

    pltpu.sync_copy(w_hbm, w_v)
    pltpu.sync_copy(utail_hbm, utail_v)
    pltpu.sync_copy(itail_hbm, itail_v)
    wd = plsc.load_gather(w_v, [iota * 0 + dg])

    for ch, (ioff, isz) in enumerate(((0, ICH0), (ICH0, ICH1), (None, 32))):
        if ioff is not None:
            pltpu.sync_copy(it_hbm.at[dg, pl.ds(ioff, isz)],
                            stage_v.at[pl.ds(0, isz)])
        for p in range(4):
            pltpu.sync_copy(i_idx_hbm.at[pl.ds(p * 4096, 4096)], piece_v)

            def qbody(g, carry, ch=ch, p=p, ioff=ioff, isz=isz):
                sl = pl.ds(g * LANES, LANES)
                i = piece_v[sl]
                if ioff is not None:
                    iloc = i - ioff
                    m = jnp.logical_and(iloc >= 0, iloc < isz)
                    ilc = jnp.clip(iloc, 0, isz - 1)
                    v = plsc.load_gather(stage_v, [ilc]) * wd
                else:
                    iloc = i - I_MAIN
                    m = iloc >= 0
                    ilc = dg * 32 + jnp.clip(iloc, 0, 31)
                    v = plsc.load_gather(itail_v, [ilc]) * wd
                row = p * 32 + jnp.right_shift(g, 3)
                col = jnp.bitwise_and(g, 7) * LANES
                if ch == 0:
                    q_v[row, pl.ds(col, LANES)] = jnp.where(m, v, 0.0)
                else:
                    q_v[row, pl.ds(col, LANES)] = (
                        q_v[row, pl.ds(col, LANES)] + jnp.where(m, v, 0.0))
                return carry

            lax.fori_loop(0, 256, qbody, 0)
    pltpu.sync_copy(q_v, q_sp.at[t])

    pltpu.sync_copy(u_idx_hbm, idx_v)
    lo = t * SUB
    n_mine = jnp.where(t == NTILES - 1, NUM_USERS - LAST_LO, SUB)

    def lbody(g, off):
        sl = pl.ds(g * LANES, LANES)
        u = idx_v[sl]
        uloc = u - lo
        m = jnp.logical_and(uloc >= 0, uloc < n_mine)
        b = g * LANES + iota
        packed = jnp.left_shift(uloc, 14) + b
        plsc.store_compressed(idx_v.at[pl.ds(off, LANES)], packed, mask=m)
        return off + jnp.sum(jnp.where(m, 1, 0))

    nh = lax.fori_loop(0, NGROUPS, lbody, 0)
    ngr = lax.div(nh + LANES - 1, LANES)

    def zbody(g, carry):
        acc_v[jnp.right_shift(g, 3),
              pl.ds(jnp.bitwise_and(g, 7) * LANES, LANES)] = (
            jnp.zeros((LANES,), jnp.float32))
        return carry

    lax.fori_loop(0, NGROUPS, zbody, 0)
    plsc.subcore_barrier()

    def acc_pass(clo, chn, src_ref, base, blim, ngr_):
        def fbody(g, carry2):
            sl = pl.ds(g * LANES, LANES)
            packed = idx_v[sl]
            valid = (g * LANES + iota) < nh
            uloc = jnp.right_shift(packed, 14)
            b = jnp.bitwise_and(packed, 16383)
            m = jnp.logical_and(valid,
                                jnp.logical_and(uloc >= clo,
                                                uloc < clo + chn))
            ulc = base + jnp.clip(uloc - clo, 0, blim)
            uval = plsc.load_gather(src_ref, [ulc])
            qval = plsc.load_gather(
                q_v, [jnp.right_shift(b, 7), jnp.bitwise_and(b, 127)])
            plsc.addupdate_scatter(
                acc_v, [jnp.right_shift(b, 7), jnp.bitwise_and(b, 127)],
                uval * qval, mask=m)
            return carry2

        lax.fori_loop(0, ngr_, fbody, 0)

    def d_body(dl, carry):
        dgb = c * NTILES + dl
        pltpu.sync_copy(q_sp.at[dl], q_v)
        for ch in range(2):
            sz = jnp.where(jnp.logical_and(t == NTILES - 1, ch == 1),
                           LAST_C1, SUBC)

            @pl.when(t < NTILES - 1)
            def _():
                pltpu.sync_copy(
                    ut_hbm.at[dgb, pl.ds(lo + ch * SUBC, SUBC)],
                    stage_v.at[pl.ds(0, SUBC)])

            @pl.when(t == NTILES - 1)
            def _():
                if ch == 0:
                    pltpu.sync_copy(
                        ut_hbm.at[dgb, pl.ds(LAST_LO, SUBC)],
                        stage_v.at[pl.ds(0, SUBC)])
                else:
                    pltpu.sync_copy(
                        ut_hbm.at[dgb, pl.ds(LAST_LO + SUBC, LAST_C1)],
                        stage_v.at[pl.ds(0, LAST_C1)])

            acc_pass(ch * SUBC, sz, stage_v, 0, BIG - 1, ngr)

        @pl.when(t == NTILES - 1)
        def _():
            acc_pass(U_MAIN - LAST_LO, 64, utail_v, dgb * 64, 63, ngr)
        return carry

    lax.fori_loop(0, NTILES, d_body, 0)
    pltpu.sync_copy(acc_v, part_hbm.at[dg])


def _k2_body(part_hbm, w_hbm, out_hbm, buf_v, w_v, out_v):
    c = lax.axis_index("c")
    s = lax.axis_index("s")
    wid = s * 2 + c
    iota = lax.iota(jnp.int32, LANES)
    pltpu.sync_copy(w_hbm, w_v)
    bias_v = plsc.load_gather(w_v, [iota * 0 + 32])

    @pl.when(wid < NTILES)
    def _():
        pltpu.sync_copy(part_hbm.at[:, pl.ds(wid * 8, 8), :], buf_v)

        def g(gi, carry):
            row = jnp.right_shift(gi, 3)
            cds = pl.ds(jnp.bitwise_and(gi, 7) * LANES, LANES)
            x = bias_v
            for r in range(32):
                x = x + buf_v[r, row, cds]
            out_v[row, cds] = 1.0 / (1.0 + jnp.exp(-x))
            return carry

        lax.fori_loop(0, 64, g, 0)
        pltpu.sync_copy(out_v, out_hbm.at[pl.ds(wid * 8, 8), :])


@jax.jit
def _gmf(ut, it, utail, itail, u_idx, i_idx, params):
    mesh = plsc.VectorSubcoreMesh(core_axis_name="c", subcore_axis_name="s")
    cp = pltpu.CompilerParams(
        needs_layout_passes=False, use_tc_tiling_on_sc=True)
    k1 = pl.kernel(
        _k1_body,
        out_type=jax.ShapeDtypeStruct((32, 128, 128), jnp.float32),
        mesh=mesh,
        scratch_types=[
            pltpu.VMEM((BIG,), jnp.float32),
            pltpu.VMEM((128, 128), jnp.float32),
            pltpu.VMEM((128, 128), jnp.float32),
            pltpu.VMEM((BATCH,), jnp.int32),
            pltpu.VMEM((4096,), jnp.int32),
            pltpu.VMEM((128,), jnp.float32),
            pltpu.VMEM((2048,), jnp.float32),
            pltpu.VMEM((1024,), jnp.float32),
            pltpu.VMEM_SHARED((NTILES, 128, 128), jnp.float32),
        ],
        compiler_params=cp,
    )
    part = k1(ut, it, utail, itail, u_idx, i_idx, params)
    k2 = pl.kernel(
        _k2_body,
        out_type=jax.ShapeDtypeStruct((128, 128), jnp.float32),
        mesh=mesh,
        scratch_types=[
            pltpu.VMEM((32, 8, 128), jnp.float32),
            pltpu.VMEM((128,), jnp.float32),
            pltpu.VMEM((8, 128), jnp.float32),
        ],
        compiler_params=cp,
    )
    return k2(part, params).reshape(BATCH)


def kernel(user_indices, item_indices, user_table, item_table, affine_w,
           affine_b):
    w = affine_w.reshape(-1).astype(jnp.float32)
    params = jnp.concatenate([
        w,
        jnp.broadcast_to(affine_b.astype(jnp.float32), (LANES,)),
        jnp.zeros((128 - 48,), jnp.float32),
    ])
    utail = user_table[U_MAIN:, :].T.reshape(-1)
    itail = item_table[I_MAIN:, :].T.reshape(-1)
    return _gmf(user_table.T, item_table.T, utail, itail,
                user_indices.astype(jnp.int32),
                item_indices.astype(jnp.int32), params)


# speedup vs baseline: 2.4435x; 1.0001x over previous
import jax
import jax.numpy as jnp
from jax import lax
from jax.experimental import pallas as pl
from jax.experimental.pallas import tpu as pltpu
from jax.experimental.pallas import tpu_sc as plsc

BATCH = 16384
NUM_USERS = 1000000
NUM_ITEMS = 100000
LANES = 16
NTILES = 16
NGROUPS = BATCH // LANES
U_MAIN = 999936
I_MAIN = 99968
SUB = 62464
SUBC = SUB // 2
LAST_LO = 15 * SUB
LAST_C1 = U_MAIN - LAST_LO - SUBC
ICH0 = 50048
ICH1 = I_MAIN - ICH0
BIG = 50048


def _k1_body(ut_hbm, it_hbm, utail_hbm, itail_hbm, u_idx_hbm, i_idx_hbm,
             w_hbm, part_hbm,
             stage_v, q_v, acc_v, idx_v, piece_v, w_v, utail_v, itail_v,
             q_sp):
    c = lax.axis_index("c")
    t = lax.axis_index("s")
    dg = c * NTILES + t
    iota = lax.iota(jnp.int32, LANES)

    pltpu.sync_copy(w_hbm, w_v)
    pltpu.sync_copy(utail_hbm, utail_v)
    pltpu.sync_copy(itail_hbm, itail_v)
    wd = plsc.load_gather(w_v, [iota * 0 + dg])

    for ch, (ioff, isz) in enumerate(((0, ICH0), (ICH0, ICH1), (None, 32))):
        if ioff is not None:
            pltpu.sync_copy(it_hbm.at[dg, pl.ds(ioff, isz)],
                            stage_v.at[pl.ds(0, isz)])
        for p in range(4):
            pltpu.sync_copy(i_idx_hbm.at[pl.ds(p * 4096, 4096)], piece_v)

            def qbody(g, carry, ch=ch, p=p, ioff=ioff, isz=isz):
                sl = pl.ds(g * LANES, LANES)
                i = piece_v[sl]
                if ioff is not None:
                    iloc = i - ioff
                    m = jnp.logical_and(iloc >= 0, iloc < isz)
                    ilc = jnp.clip(iloc, 0, isz - 1)
                    v = plsc.load_gather(stage_v, [ilc]) * wd
                else:
                    iloc = i - I_MAIN
                    m = iloc >= 0
                    ilc = dg * 32 + jnp.clip(iloc, 0, 31)
                    v = plsc.load_gather(itail_v, [ilc]) * wd
                row = p * 32 + jnp.right_shift(g, 3)
                col = jnp.bitwise_and(g, 7) * LANES
                if ch == 0:
                    q_v[row, pl.ds(col, LANES)] = jnp.where(m, v, 0.0)
                else:
                    q_v[row, pl.ds(col, LANES)] = (
                        q_v[row, pl.ds(col, LANES)] + jnp.where(m, v, 0.0))
                return carry

            lax.fori_loop(0, 256, qbody, 0, unroll=8)
    pltpu.sync_copy(q_v, q_sp.at[t])

    pltpu.sync_copy(u_idx_hbm, idx_v)
    lo = t * SUB
    n_mine = jnp.where(t == NTILES - 1, NUM_USERS - LAST_LO, SUB)

    def lbody(g, off):
        sl = pl.ds(g * LANES, LANES)
        u = idx_v[sl]
        uloc = u - lo
        m = jnp.logical_and(uloc >= 0, uloc < n_mine)
        b = g * LANES + iota
        packed = jnp.left_shift(uloc, 14) + b
        plsc.store_compressed(idx_v.at[pl.ds(off, LANES)], packed, mask=m)
        return off + jnp.sum(jnp.where(m, 1, 0))

    nh = lax.fori_loop(0, NGROUPS, lbody, 0)
    ngr8 = lax.div(nh + 8 * LANES - 1, 8 * LANES)

    def zbody(g, carry):
        acc_v[jnp.right_shift(g, 3),
              pl.ds(jnp.bitwise_and(g, 7) * LANES, LANES)] = (
            jnp.zeros((LANES,), jnp.float32))
        return carry

    lax.fori_loop(0, NGROUPS, zbody, 0, unroll=8)
    plsc.subcore_barrier()

    def acc_pass(clo, chn, src_ref, base, blim, ngr8_):
        def fbody(g8, carry2):
            for k in range(8):
                g = g8 * 8 + k
                sl = pl.ds(g * LANES, LANES)
                packed = idx_v[sl]
                valid = (g * LANES + iota) < nh
                uloc = jnp.right_shift(packed, 14)
                b = jnp.bitwise_and(packed, 16383)
                m = jnp.logical_and(valid,
                                    jnp.logical_and(uloc >= clo,
                                                    uloc < clo + chn))
                ulc = base + jnp.clip(uloc - clo, 0, blim)
                uval = plsc.load_gather(src_ref, [ulc])
                qval = plsc.load_gather(
                    q_v, [jnp.right_shift(b, 7), jnp.bitwise_and(b, 127)])
                plsc.addupdate_scatter(
                    acc_v, [jnp.right_shift(b, 7), jnp.bitwise_and(b, 127)],
                    uval * qval, mask=m)
            return carry2

        lax.fori_loop(0, ngr8_, fbody, 0)

    def d_body(dl, carry):
        dgb = c * NTILES + dl
        pltpu.sync_copy(q_sp.at[dl], q_v)
        for ch in range(2):
            sz = jnp.where(jnp.logical_and(t == NTILES - 1, ch == 1),
                           LAST_C1, SUBC)

            @pl.when(t < NTILES - 1)
            def _():
                pltpu.sync_copy(
                    ut_hbm.at[dgb, pl.ds(lo + ch * SUBC, SUBC)],
                    stage_v.at[pl.ds(0, SUBC)])

            @pl.when(t == NTILES - 1)
            def _():
                if ch == 0:
                    pltpu.sync_copy(
                        ut_hbm.at[dgb, pl.ds(LAST_LO, SUBC)],
                        stage_v.at[pl.ds(0, SUBC)])
                else:
                    pltpu.sync_copy(
                        ut_hbm.at[dgb, pl.ds(LAST_LO + SUBC, LAST_C1)],
                        stage_v.at[pl.ds(0, LAST_C1)])

            acc_pass(ch * SUBC, sz, stage_v, 0, BIG - 1, ngr8)

        @pl.when(t == NTILES - 1)
        def _():
            acc_pass(U_MAIN - LAST_LO, 64, utail_v, dgb * 64, 63, ngr8)
        return carry

    lax.fori_loop(0, NTILES, d_body, 0)
    pltpu.sync_copy(acc_v, part_hbm.at[dg])


def _k2_body(part_hbm, w_hbm, out_hbm, buf_v, w_v, out_v):
    c = lax.axis_index("c")
    s = lax.axis_index("s")
    wid = s * 2 + c
    iota = lax.iota(jnp.int32, LANES)
    pltpu.sync_copy(w_hbm, w_v)
    bias_v = plsc.load_gather(w_v, [iota * 0 + 32])

    @pl.when(wid < NTILES)
    def _():
        pltpu.sync_copy(part_hbm.at[:, pl.ds(wid * 8, 8), :], buf_v)

        def g(gi, carry):
            row = jnp.right_shift(gi, 3)
            cds = pl.ds(jnp.bitwise_and(gi, 7) * LANES, LANES)
            x = bias_v
            for r in range(32):
                x = x + buf_v[r, row, cds]
            out_v[row, cds] = 1.0 / (1.0 + jnp.exp(-x))
            return carry

        lax.fori_loop(0, 64, g, 0, unroll=4)
        pltpu.sync_copy(out_v, out_hbm.at[pl.ds(wid * 8, 8), :])


@jax.jit
def _gmf(ut, it, utail, itail, u_idx, i_idx, params):
    mesh = plsc.VectorSubcoreMesh(core_axis_name="c", subcore_axis_name="s")
    cp = pltpu.CompilerParams(
        needs_layout_passes=False, use_tc_tiling_on_sc=True)
    k1 = pl.kernel(
        _k1_body,
        out_type=jax.ShapeDtypeStruct((32, 128, 128), jnp.float32),
        mesh=mesh,
        scratch_types=[
            pltpu.VMEM((BIG,), jnp.float32),
            pltpu.VMEM((128, 128), jnp.float32),
            pltpu.VMEM((128, 128), jnp.float32),
            pltpu.VMEM((BATCH,), jnp.int32),
            pltpu.VMEM((4096,), jnp.int32),
            pltpu.VMEM((128,), jnp.float32),
            pltpu.VMEM((2048,), jnp.float32),
            pltpu.VMEM((1024,), jnp.float32),
            pltpu.VMEM_SHARED((NTILES, 128, 128), jnp.float32),
        ],
        compiler_params=cp,
    )
    part = k1(ut, it, utail, itail, u_idx, i_idx, params)
    k2 = pl.kernel(
        _k2_body,
        out_type=jax.ShapeDtypeStruct((128, 128), jnp.float32),
        mesh=mesh,
        scratch_types=[
            pltpu.VMEM((32, 8, 128), jnp.float32),
            pltpu.VMEM((128,), jnp.float32),
            pltpu.VMEM((8, 128), jnp.float32),
        ],
        compiler_params=cp,
    )
    return k2(part, params).reshape(BATCH)


def kernel(user_indices, item_indices, user_table, item_table, affine_w,
           affine_b):
    w = affine_w.reshape(-1).astype(jnp.float32)
    params = jnp.concatenate([
        w,
        jnp.broadcast_to(affine_b.astype(jnp.float32), (LANES,)),
        jnp.zeros((128 - 48,), jnp.float32),
    ])
    utail = user_table[U_MAIN:, :].T.reshape(-1)
    itail = item_table[I_MAIN:, :].T.reshape(-1)
    return _gmf(user_table.T, item_table.T, utail, itail,
                user_indices.astype(jnp.int32),
                item_indices.astype(jnp.int32), params)


# speedup vs baseline: 2.6680x; 1.0919x over previous
import jax
import jax.numpy as jnp
from jax import lax
from jax.experimental import pallas as pl
from jax.experimental.pallas import tpu as pltpu
from jax.experimental.pallas import tpu_sc as plsc

BATCH = 16384
NUM_USERS = 1000000
NUM_ITEMS = 100000
LANES = 16
NTILES = 16
NGROUPS = BATCH // LANES
U_MAIN = 999936
I_MAIN = 99968
SUB = 62464
SUBC = SUB // 2
LAST_LO = 15 * SUB
LAST_C1 = U_MAIN - LAST_LO - SUBC
ICHUNKS = ((0, 25088), (25088, 24960), (50048, 24960), (75008, 24960))


def _k1_body(ut_hbm, it_hbm, utail_hbm, itail_hbm, u_idx_hbm, i_idx_hbm,
             w_hbm, part_hbm,
             sbuf0, sbuf1, q_v, acc_v, idx_v, w_v, utail_v, itail_v,
             q_sp, usem, isem, qsem):
    c = lax.axis_index("c")
    t = lax.axis_index("s")
    dg = c * NTILES + t
    iota = lax.iota(jnp.int32, LANES)

    pltpu.sync_copy(w_hbm, w_v)
    pltpu.sync_copy(utail_hbm.at[pl.ds(c * 1024, 1024)], utail_v)
    pltpu.sync_copy(itail_hbm, itail_v)
    wd = plsc.load_gather(w_v, [iota * 0 + dg])

    pltpu.sync_copy(i_idx_hbm, idx_v)

    def i_stage(ch):
        ioff, isz = ICHUNKS[ch]
        return (it_hbm.at[dg, pl.ds(ioff, isz)],
                (sbuf0 if ch % 2 == 0 else sbuf1).at[pl.ds(0, isz)])

    pltpu.async_copy(*i_stage(0), isem)
    for ch in range(4):
        if ch < 3:
            pltpu.async_copy(*i_stage(ch + 1), isem)
        pltpu.make_async_copy(*i_stage(ch), isem).wait()
        ioff, isz = ICHUNKS[ch]

        def qbody(g, carry, ch=ch, ioff=ioff, isz=isz):
            sl = pl.ds(g * LANES, LANES)
            i = idx_v[sl]
            iloc = i - ioff
            m = jnp.logical_and(iloc >= 0, iloc < isz)
            ilc = jnp.clip(iloc, 0, isz - 1)
            v = plsc.load_gather(sbuf0 if ch % 2 == 0 else sbuf1, [ilc]) * wd
            row = jnp.right_shift(g, 3)
            col = jnp.bitwise_and(g, 7) * LANES
            if ch == 0:
                q_v[row, pl.ds(col, LANES)] = jnp.where(m, v, 0.0)
            else:
                q_v[row, pl.ds(col, LANES)] = (
                    q_v[row, pl.ds(col, LANES)] + jnp.where(m, v, 0.0))
            return carry

        lax.fori_loop(0, NGROUPS, qbody, 0, unroll=8)

    def qtail(g, carry):
        sl = pl.ds(g * LANES, LANES)
        i = idx_v[sl]
        iloc = i - I_MAIN
        m = iloc >= 0
        ilc = dg * 32 + jnp.clip(iloc, 0, 31)
        v = plsc.load_gather(itail_v, [ilc]) * wd
        row = jnp.right_shift(g, 3)
        col = jnp.bitwise_and(g, 7) * LANES
        q_v[row, pl.ds(col, LANES)] = (
            q_v[row, pl.ds(col, LANES)] + jnp.where(m, v, 0.0))
        return carry

    lax.fori_loop(0, NGROUPS, qtail, 0, unroll=8)
    pltpu.sync_copy(q_v, q_sp.at[t])

    pltpu.sync_copy(u_idx_hbm, idx_v)
    lo = t * SUB
    n_mine = jnp.where(t == NTILES - 1, NUM_USERS - LAST_LO, SUB)

    def lbody(g, off):
        sl = pl.ds(g * LANES, LANES)
        u = idx_v[sl]
        uloc = u - lo
        m = jnp.logical_and(uloc >= 0, uloc < n_mine)
        b = g * LANES + iota
        packed = jnp.left_shift(uloc, 14) + b
        plsc.store_compressed(idx_v.at[pl.ds(off, LANES)], packed, mask=m)
        return off + jnp.sum(jnp.where(m, 1, 0))

    nh = lax.fori_loop(0, NGROUPS, lbody, 0)
    ngr8 = lax.div(nh + 8 * LANES - 1, 8 * LANES)

    def zbody(g, carry):
        acc_v[jnp.right_shift(g, 3),
              pl.ds(jnp.bitwise_and(g, 7) * LANES, LANES)] = (
            jnp.zeros((LANES,), jnp.float32))
        return carry

    lax.fori_loop(0, NGROUPS, zbody, 0, unroll=8)
    plsc.subcore_barrier()

    def acc_pass(clo, chn, src_ref, base, blim, ngr8_):
        def fbody(g8, carry2):
            for k in range(8):
                g = g8 * 8 + k
                sl = pl.ds(g * LANES, LANES)
                packed = idx_v[sl]
                valid = (g * LANES + iota) < nh
                uloc = jnp.right_shift(packed, 14)
                b = jnp.bitwise_and(packed, 16383)
                m = jnp.logical_and(valid,
                                    jnp.logical_and(uloc >= clo,
                                                    uloc < clo + chn))
                ulc = base + jnp.clip(uloc - clo, 0, blim)
                uval = plsc.load_gather(src_ref, [ulc])
                qval = plsc.load_gather(
                    q_v, [jnp.right_shift(b, 7), jnp.bitwise_and(b, 127)])
                plsc.addupdate_scatter(
                    acc_v, [jnp.right_shift(b, 7), jnp.bitwise_and(b, 127)],
                    uval * qval, mask=m)
            return carry2

        lax.fori_loop(0, ngr8_, fbody, 0)

    def u_stage(dgb, ch):
        return (ut_hbm.at[dgb, pl.ds(lo + ch * SUBC, SUBC)],
                (sbuf0 if ch == 0 else sbuf1).at[pl.ds(0, SUBC)])

    def u_stage15(dgb):
        return (ut_hbm.at[dgb, pl.ds(LAST_LO + SUBC, LAST_C1)],
                sbuf1.at[pl.ds(0, LAST_C1)])

    def d_body(dl, carry):
        dgb = c * NTILES + dl
        qc = pltpu.async_copy(q_sp.at[dl], q_v, qsem)
        pltpu.async_copy(*u_stage(dgb, 0), usem)

        @pl.when(t < NTILES - 1)
        def _():
            pltpu.async_copy(*u_stage(dgb, 1), usem)

        @pl.when(t == NTILES - 1)
        def _():
            pltpu.async_copy(*u_stage15(dgb), usem)

        qc.wait()
        pltpu.make_async_copy(*u_stage(dgb, 0), usem).wait()
        acc_pass(0, SUBC, sbuf0, 0, SUBC - 1, ngr8)

        sz1 = jnp.where(t == NTILES - 1, LAST_C1, SUBC)

        @pl.when(t < NTILES - 1)
        def _():
            pltpu.make_async_copy(*u_stage(dgb, 1), usem).wait()

        @pl.when(t == NTILES - 1)
        def _():
            pltpu.make_async_copy(*u_stage15(dgb), usem).wait()

        acc_pass(SUBC, sz1, sbuf1, 0, SUBC - 1, ngr8)

        @pl.when(t == NTILES - 1)
        def _():
            acc_pass(U_MAIN - LAST_LO, 64, utail_v, dl * 64, 63, ngr8)
        return carry

    lax.fori_loop(0, NTILES, d_body, 0)
    pltpu.sync_copy(acc_v, part_hbm.at[dg])


def _k2_body(part_hbm, w_hbm, out_hbm, buf_v, w_v, out_v):
    c = lax.axis_index("c")
    s = lax.axis_index("s")
    wid = s * 2 + c
    iota = lax.iota(jnp.int32, LANES)
    pltpu.sync_copy(w_hbm, w_v)
    bias_v = plsc.load_gather(w_v, [iota * 0 + 32])

    @pl.when(wid < NTILES)
    def _():
        pltpu.sync_copy(part_hbm.at[:, pl.ds(wid * 8, 8), :], buf_v)

        def g(gi, carry):
            row = jnp.right_shift(gi, 3)
            cds = pl.ds(jnp.bitwise_and(gi, 7) * LANES, LANES)
            x = bias_v
            for r in range(32):
                x = x + buf_v[r, row, cds]
            out_v[row, cds] = 1.0 / (1.0 + jnp.exp(-x))
            return carry

        lax.fori_loop(0, 64, g, 0, unroll=4)
        pltpu.sync_copy(out_v, out_hbm.at[pl.ds(wid * 8, 8), :])


@jax.jit
def _gmf(ut, it, utail, itail, u_idx, i_idx, params):
    mesh = plsc.VectorSubcoreMesh(core_axis_name="c", subcore_axis_name="s")
    cp = pltpu.CompilerParams(
        needs_layout_passes=False, use_tc_tiling_on_sc=True)
    k1 = pl.kernel(
        _k1_body,
        out_type=jax.ShapeDtypeStruct((32, 128, 128), jnp.float32),
        mesh=mesh,
        scratch_types=[
            pltpu.VMEM((SUBC,), jnp.float32),
            pltpu.VMEM((SUBC,), jnp.float32),
            pltpu.VMEM((128, 128), jnp.float32),
            pltpu.VMEM((128, 128), jnp.float32),
            pltpu.VMEM((BATCH,), jnp.int32),
            pltpu.VMEM((128,), jnp.float32),
            pltpu.VMEM((1024,), jnp.float32),
            pltpu.VMEM((1024,), jnp.float32),
            pltpu.VMEM_SHARED((NTILES, 128, 128), jnp.float32),
            pltpu.SemaphoreType.DMA,
            pltpu.SemaphoreType.DMA,
            pltpu.SemaphoreType.DMA,
        ],
        compiler_params=cp,
    )
    part = k1(ut, it, utail, itail, u_idx, i_idx, params)
    k2 = pl.kernel(
        _k2_body,
        out_type=jax.ShapeDtypeStruct((128, 128), jnp.float32),
        mesh=mesh,
        scratch_types=[
            pltpu.VMEM((32, 8, 128), jnp.float32),
            pltpu.VMEM((128,), jnp.float32),
            pltpu.VMEM((8, 128), jnp.float32),
        ],
        compiler_params=cp,
    )
    return k2(part, params).reshape(BATCH)


def kernel(user_indices, item_indices, user_table, item_table, affine_w,
           affine_b):
    w = affine_w.reshape(-1).astype(jnp.float32)
    params = jnp.concatenate([
        w,
        jnp.broadcast_to(affine_b.astype(jnp.float32), (LANES,)),
        jnp.zeros((128 - 48,), jnp.float32),
    ])
    utail = user_table[U_MAIN:, :].T.reshape(-1)
    itail = item_table[I_MAIN:, :].T.reshape(-1)
    return _gmf(user_table.T, item_table.T, utail, itail,
                user_indices.astype(jnp.int32),
                item_indices.astype(jnp.int32), params)
